# baseline (speedup 1.0000x reference)
import functools

import jax
import jax.numpy as jnp
from jax import lax
from jax.experimental import pallas as pl
from jax.experimental.pallas import tpu as pltpu
from jax.experimental.pallas import tpu_sc as plsc

FIBONACCI = (1, 1, 2, 3, 5, 8, 13, 21)
HASH_PRIMES = (2654435761, 2246822519, 3266489917, 2028178513, 1220703125,
               1610612741, 805306457, 402653189, 3674653429, 2860486313,
               1073676287, 2971215073, 1500450271, 3267000013, 2654435789,
               4049292737, 2246822531, 3266489927, 2028178519, 1220703133,
               1610612743, 805306459, 402653191, 3674653433, 2654435771,
               2246822527, 3266489933, 2028178529, 1220703137, 1610612747,
               805306463, 402653197)

B, T = 256, 512
VOCAB = 1024
BYTE_DIM = 128
NUM_TABLES = 8
BUCKETS = 32768
EMBED = 16

NC, NS = 2, 16
NW = NC * NS
CHUNK = 128
NCHUNKS = (B * T) // CHUNK
CHUNKS_PER_W = NCHUNKS // NW
VPW = VOCAB // NW

_P32 = tuple(int(jnp.int32(jnp.uint32(HASH_PRIMES[(3 * i) % 32] & 0xFFFFFFFF)))
             if HASH_PRIMES[(3 * i) % 32] < 2**31
             else HASH_PRIMES[(3 * i) % 32] - 2**32
             for i in range(NUM_TABLES))

_MESH = plsc.VectorSubcoreMesh(core_axis_name="c", subcore_axis_name="s")
_PARAMS = pltpu.CompilerParams(use_tc_tiling_on_sc=False)


def _wid():
    return lax.axis_index("s") * NC + lax.axis_index("c")


@functools.partial(
    pl.kernel,
    out_type=jax.ShapeDtypeStruct((NUM_TABLES * VOCAB, EMBED), jnp.float32),
    mesh=_MESH,
    scratch_types=[
        pltpu.VMEM((2, 128), jnp.int32),
        pltpu.VMEM((NUM_TABLES * VPW, EMBED), jnp.float32),
        pltpu.SemaphoreType.DMA,
    ],
    compiler_params=_PARAMS,
)
def _build_fused_table(hash_flat, h_out, idx_v, rows_v, sem):
    wid = _wid()
    vbase = pl.multiple_of(wid * VPW, VPW)
    for i in range(NUM_TABLES):
        for j in range(VPW // 16):
            v = lax.iota(jnp.int32, 16) + (vbase + j * 16)
            bucket = (v * jnp.int32(_P32[i])) & jnp.int32(0x7FFF)
            gidx = bucket + jnp.int32(i * BUCKETS)
            pos = i * VPW + j * 16
            idx_v[pos // 128, pl.ds(pos % 128, 16)] = gidx
    half = (NUM_TABLES * VPW) // 2
    pltpu.async_copy(hash_flat.at[idx_v.at[jnp.int32(0)]],
                     rows_v.at[pl.ds(0, half)], sem).wait()
    pltpu.async_copy(hash_flat.at[idx_v.at[jnp.int32(1)]],
                     rows_v.at[pl.ds(half, half)], sem).wait()
    for i in range(NUM_TABLES):
        dst = pl.multiple_of(i * VOCAB + wid * VPW, VPW)
        pltpu.sync_copy(rows_v.at[pl.ds(i * VPW, VPW)],
                        h_out.at[pl.ds(dst, VPW)])


@functools.partial(
    pl.kernel,
    out_type=jax.ShapeDtypeStruct((NCHUNKS, CHUNK, BYTE_DIM + NUM_TABLES * EMBED),
                                  jnp.float32),
    mesh=_MESH,
    scratch_types=[
        pltpu.VMEM((3, 1 + NUM_TABLES, CHUNK), jnp.int32),
        pltpu.VMEM((3, 160), jnp.int32),
        pltpu.VMEM((3, CHUNK, BYTE_DIM), jnp.float32),
        pltpu.VMEM((3, NUM_TABLES, CHUNK, EMBED), jnp.float32),
        pltpu.SemaphoreType.DMA((3,)),
        pltpu.SemaphoreType.DMA((3,)),
    ],
    compiler_params=_PARAMS,
)
def _gather_embed(byte_w, fused_h, tok_pad, out4, idx_v, row_v, byte_v, tab_v,
                  sem_g, sem_w):
    wid = _wid()

    def kof(j):
        return (wid * CHUNKS_PER_W + j).astype(jnp.int32)

    def fire_gathers(j, b):
        k = kof(j)
        brow = k // jnp.int32(T // CHUNK)
        t0 = pl.multiple_of((k % jnp.int32(T // CHUNK)) * jnp.int32(CHUNK),
                            CHUNK)
        pltpu.sync_copy(tok_pad.at[brow, pl.ds(t0, 160)], row_v.at[b])
        iv = idx_v.at[b]
        rv = row_v.at[b]
        for g in range(1 + NUM_TABLES):
            off = 24 - (0 if g == 0 else FIBONACCI[g - 1])
            base = 0 if g == 0 else (g - 1) * VOCAB
            for j2 in range(CHUNK // 16):
                vec = rv[pl.ds(off + j2 * 16, 16)]
                if base:
                    vec = vec + jnp.int32(base)
                iv[g, pl.ds(j2 * 16, 16)] = vec
        pltpu.async_copy(byte_w.at[iv.at[jnp.int32(0)]], byte_v.at[b],
                         sem_g.at[b])
        for g in range(NUM_TABLES):
            pltpu.async_copy(fused_h.at[iv.at[jnp.int32(1 + g)]],
                             tab_v.at[b].at[jnp.int32(g)], sem_g.at[b])

    def wait_gathers(b):
        iv = idx_v.at[b]
        pltpu.make_async_copy(byte_w.at[iv.at[jnp.int32(0)]], byte_v.at[b],
                              sem_g.at[b]).wait()
        for g in range(NUM_TABLES):
            pltpu.make_async_copy(fused_h.at[iv.at[jnp.int32(1 + g)]],
                                  tab_v.at[b].at[jnp.int32(g)],
                                  sem_g.at[b]).wait()

    def fire_writes(j, b):
        dst = out4.at[kof(j)]
        pltpu.async_copy(byte_v.at[b], dst.at[:, pl.ds(0, BYTE_DIM)],
                         sem_w.at[b])
        for g in range(NUM_TABLES):
            pltpu.async_copy(tab_v.at[b].at[jnp.int32(g)],
                             dst.at[:, pl.ds(BYTE_DIM + g * EMBED, EMBED)],
                             sem_w.at[b])

    def wait_writes(j, b):
        dst = out4.at[kof(j)]
        pltpu.make_async_copy(byte_v.at[b], dst.at[:, pl.ds(0, BYTE_DIM)],
                              sem_w.at[b]).wait()
        for g in range(NUM_TABLES):
            pltpu.make_async_copy(tab_v.at[b].at[jnp.int32(g)],
                                  dst.at[:, pl.ds(BYTE_DIM + g * EMBED, EMBED)],
                                  sem_w.at[b]).wait()

    def body(j, carry):
        b = (j % 3).astype(jnp.int32)
        bp = ((j + 2) % 3).astype(jnp.int32)

        @pl.when(j >= 3)
        def _():
            wait_writes(j, b)

        fire_gathers(j, b)

        @pl.when(j >= 1)
        def _():
            wait_gathers(bp)
            fire_writes(j - 1, bp)

        return carry

    lax.fori_loop(jnp.int32(0), jnp.int32(CHUNKS_PER_W), body, jnp.int32(0))
    last = jnp.int32(CHUNKS_PER_W - 1)
    bl = last % 3
    wait_gathers(bl)
    fire_writes(last, bl)
    for d in range(3):
        wait_writes(last, jnp.int32(d))


def kernel(byte_embed_W, hash_tables, tokens):
    tok32 = tokens.astype(jnp.int32)
    hash_flat = hash_tables.reshape(NUM_TABLES * BUCKETS, EMBED)
    tok_pad = jnp.pad(tok32, ((0, 0), (24, 8)))

    fused_h = _build_fused_table(hash_flat)
    out4 = _gather_embed(byte_embed_W, fused_h, tok_pad)
    return out4.reshape(B, T, BYTE_DIM + NUM_TABLES * EMBED)


# speedup vs baseline: 1.3620x; 1.3620x over previous
import functools

import jax
import jax.numpy as jnp
from jax import lax
from jax.experimental import pallas as pl
from jax.experimental.pallas import tpu as pltpu
from jax.experimental.pallas import tpu_sc as plsc

FIBONACCI = (1, 1, 2, 3, 5, 8, 13, 21)
HASH_PRIMES = (2654435761, 2246822519, 3266489917, 2028178513, 1220703125,
               1610612741, 805306457, 402653189, 3674653429, 2860486313,
               1073676287, 2971215073, 1500450271, 3267000013, 2654435789,
               4049292737, 2246822531, 3266489927, 2028178519, 1220703133,
               1610612743, 805306459, 402653191, 3674653433, 2654435771,
               2246822527, 3266489933, 2028178529, 1220703137, 1610612747,
               805306463, 402653197)

B, T = 256, 512
VOCAB = 1024
BYTE_DIM = 128
NUM_TABLES = 8
BUCKETS = 32768
EMBED = 16

NC, NS = 2, 16
NW = NC * NS
CHUNK = 128
NCHUNKS = (B * T) // CHUNK
CHUNKS_PER_W = NCHUNKS // NW
NGROUPS = 1 + NUM_TABLES
VPS = VOCAB // NS

_P32 = tuple(HASH_PRIMES[(3 * i) % 32]
             if HASH_PRIMES[(3 * i) % 32] < 2**31
             else HASH_PRIMES[(3 * i) % 32] - 2**32
             for i in range(NUM_TABLES))

_MESH = plsc.VectorSubcoreMesh(core_axis_name="c", subcore_axis_name="s")
_PARAMS = pltpu.CompilerParams(use_tc_tiling_on_sc=False)


@functools.partial(
    pl.kernel,
    out_type=jax.ShapeDtypeStruct((NCHUNKS, CHUNK, BYTE_DIM + NUM_TABLES * EMBED),
                                  jnp.float32),
    mesh=_MESH,
    scratch_types=[
        pltpu.VMEM_SHARED((NUM_TABLES * VOCAB, EMBED), jnp.float32),
        pltpu.VMEM_SHARED((VOCAB, BYTE_DIM), jnp.float32),
        pltpu.VMEM((NUM_TABLES * VPS // CHUNK, CHUNK), jnp.int32),
        pltpu.VMEM((NUM_TABLES * VPS, EMBED), jnp.float32),
        pltpu.VMEM((3, NGROUPS, CHUNK), jnp.int32),
        pltpu.VMEM((3, CHUNK, BYTE_DIM), jnp.float32),
        pltpu.VMEM((3, NUM_TABLES, CHUNK, EMBED), jnp.float32),
        pltpu.SemaphoreType.DMA((3,)),
        pltpu.SemaphoreType.DMA((3,)),
        pltpu.SemaphoreType.DMA,
    ],
    compiler_params=_PARAMS,
)
def _poly_hash_embed(byte_w, hash_flat, tok9, out4,
                     h_sp, byte_sp, bidx, brows, idx_v, byte_v, tab_v,
                     sem_g, sem_w, sem_a):
    sid = lax.axis_index("s")
    wid = sid * NC + lax.axis_index("c")

    vbase = pl.multiple_of(sid * VPS, VPS)
    for i in range(NUM_TABLES):
        for j in range(VPS // 16):
            v = lax.iota(jnp.int32, 16) + (vbase + j * 16)
            bucket = (v * jnp.int32(_P32[i])) & jnp.int32(0x7FFF)
            pos = i * VPS + j * 16
            bidx[pos // CHUNK, pl.ds(pos % CHUNK, 16)] = (
                bucket + jnp.int32(i * BUCKETS))
    for c in range(NUM_TABLES * VPS // CHUNK):
        pltpu.async_copy(hash_flat.at[bidx.at[jnp.int32(c)]],
                         brows.at[pl.ds(c * CHUNK, CHUNK)], sem_a).wait()
    for i in range(NUM_TABLES):
        dst = pl.multiple_of(i * VOCAB + sid * VPS, VPS)
        pltpu.sync_copy(brows.at[pl.ds(i * VPS, VPS)],
                        h_sp.at[pl.ds(dst, VPS)])
    pltpu.sync_copy(byte_w.at[pl.ds(vbase, VPS)], byte_sp.at[pl.ds(vbase, VPS)])
    plsc.subcore_barrier()

    def kof(j):
        return (wid * CHUNKS_PER_W + j).astype(jnp.int32)

    def _copies(b, mk):
        iv = idx_v.at[b]
        out = [mk(byte_sp.at[iv.at[jnp.int32(0)]], byte_v.at[b], sem_g.at[b])]
        for g in range(NUM_TABLES):
            src = h_sp.at[pl.ds(g * VOCAB, VOCAB)]
            out.append(mk(src.at[iv.at[jnp.int32(1 + g)]],
                          tab_v.at[b].at[jnp.int32(g)], sem_g.at[b]))
        return out

    def fire_gathers(j, b):
        pltpu.sync_copy(tok9.at[kof(j)], idx_v.at[b])
        _copies(b, pltpu.async_copy)

    def wait_gathers(b):
        for cp in _copies(b, pltpu.make_async_copy):
            cp.wait()

    def fire_writes(j, b):
        dst = out4.at[kof(j)]
        pltpu.async_copy(byte_v.at[b], dst.at[:, pl.ds(0, BYTE_DIM)],
                         sem_w.at[b])
        for g in range(NUM_TABLES):
            pltpu.async_copy(tab_v.at[b].at[jnp.int32(g)],
                             dst.at[:, pl.ds(BYTE_DIM + g * EMBED, EMBED)],
                             sem_w.at[b])

    def wait_writes(j, b):
        dst = out4.at[kof(j)]
        pltpu.make_async_copy(byte_v.at[b], dst.at[:, pl.ds(0, BYTE_DIM)],
                              sem_w.at[b]).wait()
        for g in range(NUM_TABLES):
            pltpu.make_async_copy(tab_v.at[b].at[jnp.int32(g)],
                                  dst.at[:, pl.ds(BYTE_DIM + g * EMBED, EMBED)],
                                  sem_w.at[b]).wait()

    def body(j, carry):
        b = (j % 3).astype(jnp.int32)
        bp = ((j + 2) % 3).astype(jnp.int32)

        @pl.when(j >= 3)
        def _():
            wait_writes(j, b)

        fire_gathers(j, b)

        @pl.when(j >= 1)
        def _():
            wait_gathers(bp)
            fire_writes(j - 1, bp)

        return carry

    lax.fori_loop(jnp.int32(0), jnp.int32(CHUNKS_PER_W), body, jnp.int32(0))
    last = jnp.int32(CHUNKS_PER_W - 1)
    bl = last % 3
    wait_gathers(bl)
    fire_writes(last, bl)
    for d in range(3):
        wait_writes(last, jnp.int32(d))


def kernel(byte_embed_W, hash_tables, tokens):
    tok32 = tokens.astype(jnp.int32)
    hash_flat = hash_tables.reshape(NUM_TABLES * BUCKETS, EMBED)

    groups = [tok32]
    for i in range(NUM_TABLES):
        f = FIBONACCI[i]
        groups.append(jnp.pad(tok32[:, :T - f], ((0, 0), (f, 0))))
    stacked = jnp.stack(groups, axis=1)
    tok9 = (stacked.reshape(B, NGROUPS, T // CHUNK, CHUNK)
            .transpose(0, 2, 1, 3)
            .reshape(NCHUNKS, NGROUPS, CHUNK))

    out4 = _poly_hash_embed(byte_embed_W, hash_flat, tok9)
    return out4.reshape(B, T, BYTE_DIM + NUM_TABLES * EMBED)
